# baseline (speedup 1.0000x reference)
import functools

import jax
import jax.numpy as jnp
from jax import lax
from jax.experimental import pallas as pl
from jax.experimental.pallas import tpu as pltpu
from jax.experimental.pallas import tpu_sc as plsc

_NC = 2
_NS = 16
_NW = _NC * _NS
_V = 21
_VP = 24
_D = 80
_DP = 128
_C = 256
_RJ = _C // 128


@functools.partial(jax.jit, static_argnames=("B",))
def _sc_embed(idx2d, table, B):
    b_per_w = B // _NW
    n_chunks = b_per_w // _C
    assert b_per_w % _C == 0 and n_chunks % 2 == 0

    mesh = plsc.VectorSubcoreMesh(core_axis_name="c", subcore_axis_name="s")

    @functools.partial(
        pl.kernel,
        mesh=mesh,
        out_type=jax.ShapeDtypeStruct((B * _DP,), jnp.float32),
        compiler_params=pltpu.CompilerParams(
            use_tc_tiling_on_sc=False, needs_layout_passes=False
        ),
        scratch_types=[
            pltpu.VMEM((_VP * _DP,), jnp.float32),
            pltpu.VMEM((2, _RJ, 128), jnp.int32),
            pltpu.VMEM((2, _C * _DP), jnp.float32),
            pltpu.SemaphoreType.DMA,
            pltpu.SemaphoreType.DMA,
        ],
    )
    def k(idx_hbm, table_hbm, out_hbm, table_v, idx_v, rows_v, ss0, ss1):
        wid = lax.axis_index("s") * _NC + lax.axis_index("c")
        wrow0 = wid * (b_per_w // 128)
        wbase = wid * b_per_w
        npairs = n_chunks // 2

        pltpu.sync_copy(table_hbm, table_v)
        lanes128 = lax.iota(jnp.int32, 16) * _DP

        def stage_idx(g, slot):
            pltpu.sync_copy(idx_hbm.at[pl.ds(wrow0 + g * _RJ, _RJ)], idx_v.at[slot])

        def expand(slot):
            rows = rows_v.at[slot]
            for j in range(_RJ):

                def grp(q, carry):
                    tidx = idx_v.at[slot, j][pl.ds(q * 16, 16)] * _DP
                    ridx = lanes128 + (j * 128 + q * 16) * _DP
                    for c in range(_D):
                        vals = plsc.load_gather(table_v, [tidx + c])
                        plsc.store_scatter(rows, [ridx + c], vals)
                    return carry

                lax.fori_loop(0, 128 // 16, grp, 0)

        def fire_scatter(g, slot, sem):
            return pltpu.async_copy(
                rows_v.at[slot],
                out_hbm.at[pl.ds((wbase + g * _C) * _DP, _C * _DP)],
                sem,
            )

        def drain_odd_scatter():
            pltpu.make_async_copy(
                rows_v.at[1], out_hbm.at[pl.ds(wbase * _DP, _C * _DP)], ss1
            ).wait()

        def body(p, carry):
            g0 = 2 * p
            stage_idx(g0, 0)
            expand(0)
            sc0 = fire_scatter(g0, 0, ss0)

            @pl.when(p >= 1)
            def _():
                drain_odd_scatter()

            stage_idx(g0 + 1, 1)
            expand(1)
            sc0.wait()
            fire_scatter(g0 + 1, 1, ss1)
            return carry

        lax.fori_loop(0, npairs, body, 0)
        drain_odd_scatter()

    return k(idx2d, table)


def kernel(residue_types, table):
    S, T = residue_types.shape
    B = S * T
    idx2d = residue_types.reshape(B // 128, 128)
    table_flat = jnp.pad(table, ((0, _VP - _V), (0, _DP - _D))).reshape(-1)
    out = _sc_embed(idx2d, table_flat, B)
    return out.reshape(B, _DP)[:, :_D].reshape(S, T, _D)


# speedup vs baseline: 3.1814x; 3.1814x over previous
import functools

import jax
import jax.numpy as jnp
from jax import lax
from jax.experimental import pallas as pl
from jax.experimental.pallas import tpu as pltpu
from jax.experimental.pallas import tpu_sc as plsc

_NC = 2
_NS = 16
_NW = _NC * _NS
_V = 21
_NP = _V * _V
_D = 80
_DP = 128
_CP = 128


@functools.partial(jax.jit, static_argnames=("BP",))
def _sc_embed(pidx2d, tpairs, BP):
    p_per_w = BP // _NW
    n_chunks = p_per_w // _CP
    assert p_per_w % _CP == 0 and n_chunks % 2 == 0

    mesh = plsc.VectorSubcoreMesh(core_axis_name="c", subcore_axis_name="s")

    @functools.partial(
        pl.kernel,
        mesh=mesh,
        out_type=jax.ShapeDtypeStruct((BP, 2, _DP), jnp.float32),
        compiler_params=pltpu.CompilerParams(use_tc_tiling_on_sc=False),
        scratch_types=[
            pltpu.VMEM((2, 1, _CP), jnp.int32),
            pltpu.VMEM((2, _CP, 2, _DP), jnp.float32),
            pltpu.SemaphoreType.DMA,
            pltpu.SemaphoreType.DMA,
            pltpu.SemaphoreType.DMA,
            pltpu.SemaphoreType.DMA,
        ],
    )
    def k(pidx_hbm, tpairs_hbm, out_hbm, idx_v, rows_v, sg0, sg1, ss0, ss1):
        wid = lax.axis_index("s") * _NC + lax.axis_index("c")
        wrow0 = wid * (p_per_w // _CP)
        wbase = wid * p_per_w
        off = wid * _NP
        npairs = n_chunks // 2

        def stage_idx(g, slot):
            pltpu.sync_copy(pidx_hbm.at[pl.ds(wrow0 + g, 1)], idx_v.at[slot])
            for q in range(_CP // 16):
                sl = idx_v.at[slot, 0][pl.ds(q * 16, 16)]
                idx_v.at[slot, 0][pl.ds(q * 16, 16)] = sl + off

        def fire_gather(slot, sem):
            return pltpu.async_copy(
                tpairs_hbm.at[idx_v.at[slot, 0]], rows_v.at[slot], sem
            )

        def fire_scatter(g, slot, sem):
            return pltpu.async_copy(
                rows_v.at[slot], out_hbm.at[pl.ds(wbase + g * _CP, _CP)], sem
            )

        def drain_odd_scatter():
            pltpu.make_async_copy(
                rows_v.at[1], out_hbm.at[pl.ds(wbase, _CP)], ss1
            ).wait()

        def body(p, carry):
            g0 = 2 * p
            stage_idx(g0, 0)
            gcp = fire_gather(0, sg0)

            @pl.when(p >= 1)
            def _():
                drain_odd_scatter()

            gcp.wait()
            sc0 = fire_scatter(g0, 0, ss0)

            stage_idx(g0 + 1, 1)
            fire_gather(1, sg1).wait()
            sc0.wait()
            fire_scatter(g0 + 1, 1, ss1)
            return carry

        lax.fori_loop(0, npairs, body, 0)
        drain_odd_scatter()

    return k(pidx2d, tpairs)


def kernel(residue_types, table):
    S, T = residue_types.shape
    B = S * T
    BP = B // 2
    pidx = residue_types.reshape(BP, 2)
    pidx2d = (pidx[:, 0] * _V + pidx[:, 1]).reshape(BP // _CP, _CP)
    tpad = jnp.pad(table, ((0, 0), (0, _DP - _D)))
    tp = jnp.concatenate(
        [
            jnp.broadcast_to(tpad[:, None, None, :], (_V, _V, 1, _DP)),
            jnp.broadcast_to(tpad[None, :, None, :], (_V, _V, 1, _DP)),
        ],
        axis=2,
    ).reshape(_NP, 2, _DP)
    tpairs = jnp.tile(tp, (_NW, 1, 1))
    out = _sc_embed(pidx2d, tpairs, BP)
    return out.reshape(B, _DP)[:, :_D].reshape(S, T, _D)
